# baseline (speedup 1.0000x reference)
import functools

import jax
import jax.numpy as jnp
from jax import lax
from jax.experimental import pallas as pl
from jax.experimental.pallas import tpu as pltpu
from jax.experimental.pallas import tpu_sc as plsc

_NBUF = 10


def kernel(vocab_ids, table):
    bsz, hist = vocab_ids.shape
    _, d = table.shape

    info = plsc.get_sparse_core_info()
    nw = info.num_cores * info.num_subcores
    bpw = bsz // nw
    half = bpw // 4
    n_chunks = 4 * hist
    n_groups = n_chunks // _NBUF
    assert bpw * nw == bsz and half * 4 == bpw
    assert n_groups * _NBUF == n_chunks
    assert bpw <= 128

    idx = vocab_ids.astype(jnp.int32).T

    mesh = plsc.VectorSubcoreMesh(core_axis_name="c", subcore_axis_name="s")

    @functools.partial(
        pl.kernel,
        out_type=jax.ShapeDtypeStruct((hist, bsz, d), table.dtype),
        mesh=mesh,
        scratch_types=[
            pltpu.VMEM((hist, bpw), jnp.int32),
            pltpu.VMEM((_NBUF, half, d), jnp.float32),
            pltpu.SemaphoreType.DMA((_NBUF,)),
            pltpu.SemaphoreType.DMA((_NBUF,)),
        ],
        compiler_params=pltpu.CompilerParams(use_tc_tiling_on_sc=True),
    )
    def emb_lookup(idx_hbm, table_hbm, out_hbm, idx_v, bufs, gsem, wsem):
        wid = lax.axis_index("s") * info.num_cores + lax.axis_index("c")
        b0 = wid * bpw
        pltpu.sync_copy(idx_hbm.at[:, pl.ds(b0, bpw)], idx_v)

        def gather(c, b):
            return pltpu.make_async_copy(
                table_hbm.at[idx_v.at[c // 4, pl.ds((c % 4) * half, half)]],
                bufs.at[b], gsem.at[b])

        def write(c, b):
            return pltpu.make_async_copy(
                bufs.at[b],
                out_hbm.at[c // 4, pl.ds(b0 + (c % 4) * half, half)],
                wsem.at[b])

        for b in range(_NBUF):
            gather(b, b).start()

        @pl.loop(0, n_groups)
        def _(g):
            c0 = g * _NBUF
            for b in range(_NBUF):
                gather(c0 + b, b).wait()
                write(c0 + b, b).start()
            for b in range(_NBUF):
                write(c0 + b, b).wait()
                nxt = c0 + _NBUF + b

                @pl.when(nxt < n_chunks)
                def _():
                    gather(nxt, b).start()

    out = emb_lookup(idx, table)
    return out.transpose(1, 0, 2)


# speedup vs baseline: 1.0479x; 1.0479x over previous
import functools

import jax
import jax.numpy as jnp
from jax import lax
from jax.experimental import pallas as pl
from jax.experimental.pallas import tpu as pltpu
from jax.experimental.pallas import tpu_sc as plsc

_NBUF = 5


def kernel(vocab_ids, table):
    bsz, hist = vocab_ids.shape
    _, d = table.shape

    info = plsc.get_sparse_core_info()
    nw = info.num_cores * info.num_subcores
    bpw = bsz // nw
    half = bpw // 2
    n_chunks = 2 * hist
    n_groups = n_chunks // _NBUF
    assert bpw * nw == bsz and half * 2 == bpw
    assert n_groups * _NBUF == n_chunks
    assert bpw <= 128

    idx = vocab_ids.astype(jnp.int32).T

    mesh = plsc.VectorSubcoreMesh(core_axis_name="c", subcore_axis_name="s")

    @functools.partial(
        pl.kernel,
        out_type=jax.ShapeDtypeStruct((hist, bsz, d), table.dtype),
        mesh=mesh,
        scratch_types=[
            pltpu.VMEM((hist, bpw), jnp.int32),
            pltpu.VMEM((_NBUF, half, d), jnp.float32),
            pltpu.SemaphoreType.DMA((_NBUF,)),
            pltpu.SemaphoreType.DMA((_NBUF,)),
        ],
        compiler_params=pltpu.CompilerParams(use_tc_tiling_on_sc=True),
    )
    def emb_lookup(idx_hbm, table_hbm, out_hbm, idx_v, bufs, gsem, wsem):
        wid = lax.axis_index("s") * info.num_cores + lax.axis_index("c")
        b0 = wid * bpw
        pltpu.sync_copy(idx_hbm.at[:, pl.ds(b0, bpw)], idx_v)

        def gather(c, b):
            return pltpu.make_async_copy(
                table_hbm.at[idx_v.at[c // 2, pl.ds((c % 2) * half, half)]],
                bufs.at[b], gsem.at[b])

        def write(c, b):
            return pltpu.make_async_copy(
                bufs.at[b],
                out_hbm.at[c // 2, pl.ds(b0 + (c % 2) * half, half)],
                wsem.at[b])

        for b in range(_NBUF):
            gather(b, b).start()

        @pl.loop(0, n_groups)
        def _(g):
            c0 = g * _NBUF
            for b in range(_NBUF):
                gather(c0 + b, b).wait()
                write(c0 + b, b).start()
            for b in range(_NBUF):
                write(c0 + b, b).wait()
                nxt = c0 + _NBUF + b

                @pl.when(nxt < n_chunks)
                def _():
                    gather(nxt, b).start()

    out = emb_lookup(idx, table)
    return out.transpose(1, 0, 2)


# speedup vs baseline: 1.0645x; 1.0159x over previous
import functools

import jax
import jax.numpy as jnp
from jax import lax
from jax.experimental import pallas as pl
from jax.experimental.pallas import tpu as pltpu
from jax.experimental.pallas import tpu_sc as plsc

_NBUF = 10


def kernel(vocab_ids, table):
    bsz, hist = vocab_ids.shape
    _, d = table.shape

    info = plsc.get_sparse_core_info()
    nw = info.num_cores * info.num_subcores
    bpw = bsz // nw
    half = bpw // 2
    n_chunks = 2 * hist
    n_groups = n_chunks // _NBUF
    assert bpw * nw == bsz and half * 2 == bpw
    assert n_groups * _NBUF == n_chunks
    assert bpw <= 128

    idx = vocab_ids.astype(jnp.int32).T

    mesh = plsc.VectorSubcoreMesh(core_axis_name="c", subcore_axis_name="s")

    @functools.partial(
        pl.kernel,
        out_type=jax.ShapeDtypeStruct((hist, bsz, d), table.dtype),
        mesh=mesh,
        scratch_types=[
            pltpu.VMEM((hist, bpw), jnp.int32),
            pltpu.VMEM((_NBUF, half, d), jnp.float32),
            pltpu.SemaphoreType.DMA((_NBUF,)),
            pltpu.SemaphoreType.DMA((_NBUF,)),
        ],
        compiler_params=pltpu.CompilerParams(use_tc_tiling_on_sc=True),
    )
    def emb_lookup(idx_hbm, table_hbm, out_hbm, idx_v, bufs, gsem, wsem):
        wid = lax.axis_index("s") * info.num_cores + lax.axis_index("c")
        b0 = wid * bpw
        pltpu.sync_copy(idx_hbm.at[:, pl.ds(b0, bpw)], idx_v)

        def gather(c, b):
            return pltpu.make_async_copy(
                table_hbm.at[idx_v.at[c // 2, pl.ds((c % 2) * half, half)]],
                bufs.at[b], gsem.at[b])

        def write(c, b):
            return pltpu.make_async_copy(
                bufs.at[b],
                out_hbm.at[c // 2, pl.ds(b0 + (c % 2) * half, half)],
                wsem.at[b])

        for b in range(_NBUF):
            gather(b, b).start()

        @pl.loop(0, n_groups)
        def _(g):
            c0 = g * _NBUF
            for b in range(_NBUF):
                gather(c0 + b, b).wait()
                write(c0 + b, b).start()
            for b in range(_NBUF):
                write(c0 + b, b).wait()
                nxt = c0 + _NBUF + b

                @pl.when(nxt < n_chunks)
                def _():
                    gather(nxt, b).start()

    out = emb_lookup(idx, table)
    return out.transpose(1, 0, 2)


# speedup vs baseline: 1.0680x; 1.0033x over previous
import functools

import jax
import jax.numpy as jnp
from jax import lax
from jax.experimental import pallas as pl
from jax.experimental.pallas import tpu as pltpu
from jax.experimental.pallas import tpu_sc as plsc

_NBUF = 10


def kernel(vocab_ids, table):
    bsz, hist = vocab_ids.shape
    _, d = table.shape

    info = plsc.get_sparse_core_info()
    nw = info.num_cores * info.num_subcores
    bpw = bsz // nw
    half = bpw // 2
    n_chunks = 2 * hist
    n_groups = n_chunks // _NBUF
    assert bpw * nw == bsz and half * 2 == bpw
    assert n_groups * _NBUF == n_chunks
    assert bpw <= 128

    idx = vocab_ids.astype(jnp.int32).T

    mesh = plsc.VectorSubcoreMesh(core_axis_name="c", subcore_axis_name="s")

    @functools.partial(
        pl.kernel,
        out_type=jax.ShapeDtypeStruct((hist, bsz, d), table.dtype),
        mesh=mesh,
        scratch_types=[
            pltpu.VMEM((hist, bpw), jnp.int32),
            pltpu.VMEM((_NBUF, half, d), jnp.float32),
            pltpu.SemaphoreType.DMA((_NBUF,)),
            pltpu.SemaphoreType.DMA((_NBUF,)),
        ],
        compiler_params=pltpu.CompilerParams(
            use_tc_tiling_on_sc=True,
            disable_bounds_checks=True,
            disable_semaphore_checks=True,
        ),
    )
    def emb_lookup(idx_hbm, table_hbm, out_hbm, idx_v, bufs, gsem, wsem):
        wid = lax.axis_index("s") * info.num_cores + lax.axis_index("c")
        b0 = wid * bpw
        pltpu.sync_copy(idx_hbm.at[:, pl.ds(b0, bpw)], idx_v)

        def gather(c, b):
            return pltpu.make_async_copy(
                table_hbm.at[idx_v.at[c // 2, pl.ds((c % 2) * half, half)]],
                bufs.at[b], gsem.at[b])

        def write(c, b):
            return pltpu.make_async_copy(
                bufs.at[b],
                out_hbm.at[c // 2, pl.ds(b0 + (c % 2) * half, half)],
                wsem.at[b])

        for b in range(_NBUF):
            gather(b, b).start()

        @pl.loop(0, n_groups)
        def _(g):
            c0 = g * _NBUF
            for b in range(_NBUF):
                gather(c0 + b, b).wait()
                write(c0 + b, b).start()
            for b in range(_NBUF):
                write(c0 + b, b).wait()
                nxt = c0 + _NBUF + b

                @pl.when(nxt < n_chunks)
                def _():
                    gather(nxt, b).start()

    out = emb_lookup(idx, table)
    return out.transpose(1, 0, 2)
